# Initial kernel scaffold; baseline (speedup 1.0000x reference)
import jax
import jax.numpy as jnp
from jax.experimental import pallas as pl


def kernel(x, edge_index, W_pool, b_pool, W_self, W_neigh, b_conv, W1, b1, W2, b2, Wr, br):
    raise NotImplementedError("write your pallas kernel here")


# speedup vs baseline: 1.8460x; 1.8460x over previous
import functools

import jax
import jax.numpy as jnp
from jax import lax
from jax.experimental import pallas as pl
from jax.experimental.pallas import tpu as pltpu
from jax.experimental.pallas import tpu_sc as plsc

NC = 2
NS = 16
NW = NC * NS
LANES = 16
G = 32


def _round_up(a, b):
    return (a + b - 1) // b * b


def _pick_chunk(E):
    for c in range(4096, 15, -16):
        if E % c == 0:
            return c
    return 16


def _tc_pool(x_pad, W_pool, b_pool2):
    n_pad = x_pad.shape[0]
    d = x_pad.shape[1]

    def body(x_ref, w_ref, b_ref, o_ref):
        acc = jnp.dot(x_ref[...], w_ref[...], preferred_element_type=jnp.float32)
        o_ref[...] = jnp.maximum(acc + b_ref[...], 0.0)

    return pl.pallas_call(
        body,
        out_shape=jax.ShapeDtypeStruct((n_pad, d), jnp.float32),
    )(x_pad, W_pool, b_pool2)


def _sc_segment_max(h_pool, src, dst, n_nodes, npt):
    E = src.shape[0]
    d = h_pool.shape[1]
    dch = d // LANES
    chunk = _pick_chunk(E)
    nchunks = E // chunk
    nvreg = chunk // LANES
    sel_cap = chunk + 2 * G

    mesh = plsc.VectorSubcoreMesh(
        core_axis_name="c", subcore_axis_name="s", num_cores=NC, num_subcores=NS
    )

    @functools.partial(
        pl.kernel,
        out_type=jax.ShapeDtypeStruct((NW, npt, d), jnp.float32),
        mesh=mesh,
        compiler_params=pltpu.CompilerParams(needs_layout_passes=False),
        scratch_types=[
            pltpu.VMEM((npt + 1, d), jnp.float32),
            pltpu.VMEM((chunk,), jnp.int32),
            pltpu.VMEM((chunk,), jnp.int32),
            pltpu.VMEM((sel_cap,), jnp.int32),
            pltpu.VMEM((sel_cap,), jnp.int32),
            pltpu.VMEM((G, d), jnp.float32),
            pltpu.SemaphoreType.DMA,
        ],
    )
    def k(hpool_hbm, src_hbm, dst_hbm, agg_hbm,
          agg_l, dst_c, src_c, sel_s, sel_l, msg, sem):
        wid = lax.axis_index("s") * NC + lax.axis_index("c")
        lo = wid * npt
        lane = lax.iota(jnp.int32, LANES)
        zf = jnp.zeros((LANES,), jnp.float32)

        def zrow(r, _):
            for cc in range(dch):
                agg_l[r, pl.ds(cc * LANES, LANES)] = zf
            return 0
        lax.fori_loop(0, npt + 1, zrow, 0)

        def process_groups(ngroups, wp_keep):
            def grp(g, _):
                pltpu.async_copy(
                    hpool_hbm.at[sel_s.at[pl.ds(g * G, G)]], msg, sem
                ).wait()
                for half in range(G // LANES):
                    lv = sel_l[pl.ds(g * G + half * LANES, LANES)]
                    for j in range(LANES):
                        loc = jnp.max(jnp.where(lane == j, lv, 0))
                        row = half * LANES + j
                        for cc in range(dch):
                            sl = pl.ds(cc * LANES, LANES)
                            agg_l[loc, sl] = jnp.maximum(
                                agg_l[loc, sl], msg[row, sl]
                            )
                return 0
            lax.fori_loop(0, ngroups, grp, 0)
            @pl.when(ngroups > 0)
            def _():
                for half in range(G // LANES):
                    svec = sel_s[pl.ds(ngroups * G + half * LANES, LANES)]
                    lvec = sel_l[pl.ds(ngroups * G + half * LANES, LANES)]
                    sel_s[pl.ds(half * LANES, LANES)] = svec
                    sel_l[pl.ds(half * LANES, LANES)] = lvec
            return wp_keep - ngroups * G

        def per_chunk(c, wp):
            pltpu.sync_copy(dst_hbm.at[pl.ds(c * chunk, chunk)], dst_c)
            pltpu.sync_copy(src_hbm.at[pl.ds(c * chunk, chunk)], src_c)

            def sel(v, wp_in):
                dv = dst_c[pl.ds(v * LANES, LANES)]
                loc = dv - lo
                m = (loc >= 0) & (loc < npt)
                cv = plsc.cumsum(jnp.where(m, 1, 0))
                pos = wp_in + cv - 1
                sv = src_c[pl.ds(v * LANES, LANES)]
                plsc.store_scatter(sel_s, [pos], sv, mask=m)
                plsc.store_scatter(sel_l, [pos], loc, mask=m)
                return wp_in + jnp.max(cv)
            wp = lax.fori_loop(0, nvreg, sel, wp)
            return process_groups(wp // G, wp)

        wp = lax.fori_loop(0, nchunks, per_chunk, jnp.int32(0))

        @pl.when(wp > 0)
        def _():
            for half in range(G // LANES):
                pidx = wp + half * LANES + lane
                mpad = pidx < G
                plsc.store_scatter(sel_s, [pidx], jnp.zeros((LANES,), jnp.int32),
                                   mask=mpad)
                plsc.store_scatter(sel_l, [pidx],
                                   jnp.full((LANES,), npt, jnp.int32), mask=mpad)
            process_groups(1, wp)

        pltpu.sync_copy(agg_l.at[pl.ds(0, npt)], agg_hbm.at[wid])

    out = k(h_pool, src, dst)
    return out.reshape(NW * npt, d)


def _tc_readout(x_pad, agg_pad, W_self, W_neigh, b_conv2, W1, b12, W2, b22,
                Wr_pad, br_pad, n_nodes, blk):
    n_pad = x_pad.shape[0]
    d = x_pad.shape[1]
    h = W_self.shape[1]
    nblk = n_pad // blk

    def body(x_ref, a_ref, ws_ref, wn_ref, bc_ref, w1_ref, b1_ref,
             w2_ref, b2_ref, wr_ref, br_ref, o_ref, m_acc, s_acc):
        i = pl.program_id(0)

        @pl.when(i == 0)
        def _():
            m_acc[...] = jnp.full((1, h), -jnp.inf, jnp.float32)
            s_acc[...] = jnp.zeros((1, h), jnp.float32)

        hb = jnp.dot(x_ref[...], ws_ref[...], preferred_element_type=jnp.float32)
        hb += jnp.dot(a_ref[...], wn_ref[...], preferred_element_type=jnp.float32)
        hb = jnp.maximum(hb + bc_ref[...], 0.0)
        hb = jnp.maximum(
            jnp.dot(hb, w1_ref[...], preferred_element_type=jnp.float32)
            + b1_ref[...], 0.0)
        hb = jnp.maximum(
            jnp.dot(hb, w2_ref[...], preferred_element_type=jnp.float32)
            + b2_ref[...], 0.0)

        rows = i * blk + lax.broadcasted_iota(jnp.int32, (blk, 1), 0)
        valid = rows < n_nodes
        hm = jnp.where(valid, hb, -jnp.inf)
        bm = jnp.max(hm, axis=0, keepdims=True)
        m_new = jnp.maximum(m_acc[...], bm)
        contrib = jnp.where(valid, jnp.exp(hb - m_new), 0.0)
        s_acc[...] = (s_acc[...] * jnp.exp(m_acc[...] - m_new)
                      + jnp.sum(contrib, axis=0, keepdims=True))
        m_acc[...] = m_new

        @pl.when(i == nblk - 1)
        def _():
            final = 1.0 / s_acc[...]
            o_ref[...] = jnp.dot(final, wr_ref[...],
                                 preferred_element_type=jnp.float32) + br_ref[...]

    full = lambda shape: pl.BlockSpec(shape, lambda i: (0, 0))
    return pl.pallas_call(
        body,
        grid=(nblk,),
        in_specs=[
            pl.BlockSpec((blk, d), lambda i: (i, 0)),
            pl.BlockSpec((blk, d), lambda i: (i, 0)),
            full((d, h)), full((d, h)), full((1, h)),
            full((h, h)), full((1, h)),
            full((h, h)), full((1, h)),
            full((h, h)), full((1, h)),
        ],
        out_specs=full((1, h)),
        out_shape=jax.ShapeDtypeStruct((1, h), jnp.float32),
        scratch_shapes=[
            pltpu.VMEM((1, h), jnp.float32),
            pltpu.VMEM((1, h), jnp.float32),
        ],
    )(x_pad, agg_pad, W_self, W_neigh, b_conv2, W1, b12, W2, b22,
      Wr_pad, br_pad)


def kernel(x, edge_index, W_pool, b_pool, W_self, W_neigh, b_conv,
           W1, b1, W2, b2, Wr, br):
    n, d = x.shape
    h = W_self.shape[1]
    out_dim = Wr.shape[1]

    npt = _round_up((n + NW - 1) // NW, 8)
    n_pad = NW * npt
    blk = 512
    n_pad = _round_up(n_pad, blk)
    npt_total = n_pad

    x_pad = jnp.pad(x, ((0, n_pad - n), (0, 0)))
    src = edge_index[0]
    dst = edge_index[1]

    h_pool = _tc_pool(x_pad, W_pool, b_pool.reshape(1, d))
    agg = _sc_segment_max(h_pool, src, dst, n, npt)
    if agg.shape[0] != n_pad:
        agg = jnp.pad(agg, ((0, n_pad - agg.shape[0]), (0, 0)))

    Wr_pad = jnp.pad(Wr, ((0, 0), (0, h - out_dim)))
    br_pad = jnp.pad(br, ((0, h - out_dim))).reshape(1, h)
    out = _tc_readout(x_pad, agg, W_self, W_neigh, b_conv.reshape(1, h),
                      W1, b1.reshape(1, h), W2, b2.reshape(1, h),
                      Wr_pad, br_pad, n, blk)
    return out[:, :out_dim]


# speedup vs baseline: 2.0725x; 1.1227x over previous
import functools

import jax
import jax.numpy as jnp
from jax import lax
from jax.experimental import pallas as pl
from jax.experimental.pallas import tpu as pltpu
from jax.experimental.pallas import tpu_sc as plsc

NC = 2
NS = 16
NW = NC * NS
LANES = 16
G = 32


def _round_up(a, b):
    return (a + b - 1) // b * b


def _pick_chunk(E):
    for c in range(8192, 63, -64):
        if E % c == 0:
            return c
    for c in range(8192, 15, -16):
        if E % c == 0:
            return c
    return 16


def _tc_pool(x_pad, W_pool, b_pool2):
    n_pad = x_pad.shape[0]
    d = x_pad.shape[1]

    def body(x_ref, w_ref, b_ref, o_ref):
        acc = jnp.dot(x_ref[...], w_ref[...], preferred_element_type=jnp.float32)
        o_ref[...] = jnp.maximum(acc + b_ref[...], 0.0)

    return pl.pallas_call(
        body,
        out_shape=jax.ShapeDtypeStruct((n_pad, d), jnp.float32),
    )(x_pad, W_pool, b_pool2)


def _sc_segment_max(h_pool, src, dst, n_nodes, npt):
    E = src.shape[0]
    d = h_pool.shape[1]
    dch = d // LANES
    chunk = _pick_chunk(E)
    nchunks = E // chunk
    nvreg = chunk // LANES
    sel_cap = chunk + 2 * G

    mesh = plsc.VectorSubcoreMesh(
        core_axis_name="c", subcore_axis_name="s", num_cores=NC, num_subcores=NS
    )

    @functools.partial(
        pl.kernel,
        out_type=jax.ShapeDtypeStruct((NW, npt, d), jnp.float32),
        mesh=mesh,
        compiler_params=pltpu.CompilerParams(needs_layout_passes=False),
        scratch_types=[
            pltpu.VMEM((npt + 1, d), jnp.float32),
            pltpu.VMEM((chunk,), jnp.int32),
            pltpu.VMEM((chunk,), jnp.int32),
            pltpu.VMEM((chunk,), jnp.int32),
            pltpu.VMEM((chunk,), jnp.int32),
            pltpu.VMEM((sel_cap,), jnp.int32),
            pltpu.VMEM((sel_cap,), jnp.int32),
            pltpu.VMEM((G, d), jnp.float32),
            pltpu.VMEM((G, d), jnp.float32),
            pltpu.SemaphoreType.DMA,
            pltpu.SemaphoreType.DMA,
            pltpu.SemaphoreType.DMA,
            pltpu.SemaphoreType.DMA,
        ],
    )
    def k(hpool_hbm, src_hbm, dst_hbm, agg_hbm,
          agg_l, dst_c0, dst_c1, src_c0, src_c1, sel_s, sel_l, msg0, msg1,
          semc0, semc1, semg0, semg1):
        wid = lax.axis_index("s") * NC + lax.axis_index("c")
        lo = wid * npt
        lane = lax.iota(jnp.int32, LANES)
        zf = jnp.zeros((LANES,), jnp.float32)
        dst_c = [dst_c0, dst_c1]
        src_c = [src_c0, src_c1]
        msg = [msg0, msg1]
        semc = [semc0, semc1]
        semg = [semg0, semg1]

        def zrow(r, _):
            for cc in range(dch):
                agg_l[r, pl.ds(cc * LANES, LANES)] = zf
            return 0
        lax.fori_loop(0, npt + 1, zrow, 0)

        def fire_chunk(c, k_):
            pltpu.async_copy(dst_hbm.at[pl.ds(c * chunk, chunk)],
                             dst_c[k_], semc[k_])
            pltpu.async_copy(src_hbm.at[pl.ds(c * chunk, chunk)],
                             src_c[k_], semc[k_])

        def wait_chunk(k_):
            pltpu.make_async_copy(dst_hbm.at[pl.ds(0, chunk)],
                                  dst_c[k_], semc[k_]).wait()
            pltpu.make_async_copy(src_hbm.at[pl.ds(0, chunk)],
                                  src_c[k_], semc[k_]).wait()

        def fire_grp(g, b):
            pltpu.async_copy(hpool_hbm.at[sel_s.at[pl.ds(g * G, G)]],
                             msg[b], semg[b])

        def wait_grp(b):
            pltpu.make_async_copy(hpool_hbm.at[sel_s.at[pl.ds(0, G)]],
                                  msg[b], semg[b]).wait()

        def accum_grp(g, b):
            for half in range(G // LANES):
                lv = sel_l[pl.ds(g * G + half * LANES, LANES)]
                for j in range(LANES):
                    loc = jnp.max(jnp.where(lane == j, lv, 0))
                    row = half * LANES + j
                    for cc in range(dch):
                        sl = pl.ds(cc * LANES, LANES)
                        agg_l[loc, sl] = jnp.maximum(
                            agg_l[loc, sl], msg[b][row, sl]
                        )

        def process_groups(ngroups, wp_keep):
            @pl.when(ngroups > 0)
            def _():
                fire_grp(0, 0)

            def pair(p, _):
                g0 = 2 * p
                g1 = 2 * p + 1

                @pl.when(g1 < ngroups)
                def _():
                    fire_grp(g1, 1)
                wait_grp(0)
                accum_grp(g0, 0)

                @pl.when(g1 < ngroups)
                def _():
                    @pl.when(g1 + 1 < ngroups)
                    def _():
                        fire_grp(g1 + 1, 0)
                    wait_grp(1)
                    accum_grp(g1, 1)
                return 0
            lax.fori_loop(0, (ngroups + 1) // 2, pair, 0)

            @pl.when(ngroups > 0)
            def _():
                for half in range(G // LANES):
                    svec = sel_s[pl.ds(ngroups * G + half * LANES, LANES)]
                    lvec = sel_l[pl.ds(ngroups * G + half * LANES, LANES)]
                    sel_s[pl.ds(half * LANES, LANES)] = svec
                    sel_l[pl.ds(half * LANES, LANES)] = lvec
            return wp_keep - ngroups * G

        unpt = jnp.uint32(npt)

        def per_chunk(c, wp):
            k_ = lax.rem(c, 2)
            @pl.when(k_ == 0)
            def _():
                wait_chunk(0)

            @pl.when(k_ == 1)
            def _():
                wait_chunk(1)

            @pl.when((c + 1 < nchunks) & (k_ == 0))
            def _():
                fire_chunk(c + 1, 1)

            @pl.when((c + 1 < nchunks) & (k_ == 1))
            def _():
                fire_chunk(c + 1, 0)

            def sel_body(v, wp_in, kb):
                dv = dst_c[kb][pl.ds(v * LANES, LANES)]
                loc = dv - lo
                m = loc.astype(jnp.uint32) < unpt
                cv = plsc.cumsum(jnp.where(m, 1, 0))
                pos = wp_in + cv - 1
                sv = src_c[kb][pl.ds(v * LANES, LANES)]
                plsc.store_scatter(sel_s, [pos], sv, mask=m)
                plsc.store_scatter(sel_l, [pos], loc, mask=m)
                return wp_in + jnp.max(cv)

            def run_sel(wp_in, kb):
                def quad(q, w):
                    for u in range(4):
                        w = sel_body(4 * q + u, w, kb)
                    return w
                return lax.fori_loop(0, nvreg // 4, quad, wp_in)

            wp = lax.cond(k_ == 0,
                          lambda w: run_sel(w, 0),
                          lambda w: run_sel(w, 1), wp)
            return process_groups(wp // G, wp)

        fire_chunk(0, 0)
        wp = lax.fori_loop(0, nchunks, per_chunk, jnp.int32(0))

        @pl.when(wp > 0)
        def _():
            for half in range(G // LANES):
                pidx = wp + half * LANES + lane
                mpad = pidx < G
                plsc.store_scatter(sel_s, [pidx], jnp.zeros((LANES,), jnp.int32),
                                   mask=mpad)
                plsc.store_scatter(sel_l, [pidx],
                                   jnp.full((LANES,), npt, jnp.int32), mask=mpad)
            process_groups(1, wp)

        pltpu.sync_copy(agg_l.at[pl.ds(0, npt)], agg_hbm.at[wid])

    out = k(h_pool, src, dst)
    return out.reshape(NW * npt, d)


def _tc_readout(x_pad, agg_pad, W_self, W_neigh, b_conv2, W1, b12, W2, b22,
                Wr_pad, br_pad, n_nodes, blk):
    n_pad = x_pad.shape[0]
    d = x_pad.shape[1]
    h = W_self.shape[1]
    nblk = n_pad // blk

    def body(x_ref, a_ref, ws_ref, wn_ref, bc_ref, w1_ref, b1_ref,
             w2_ref, b2_ref, wr_ref, br_ref, o_ref, m_acc, s_acc):
        i = pl.program_id(0)

        @pl.when(i == 0)
        def _():
            m_acc[...] = jnp.full((1, h), -jnp.inf, jnp.float32)
            s_acc[...] = jnp.zeros((1, h), jnp.float32)

        hb = jnp.dot(x_ref[...], ws_ref[...], preferred_element_type=jnp.float32)
        hb += jnp.dot(a_ref[...], wn_ref[...], preferred_element_type=jnp.float32)
        hb = jnp.maximum(hb + bc_ref[...], 0.0)
        hb = jnp.maximum(
            jnp.dot(hb, w1_ref[...], preferred_element_type=jnp.float32)
            + b1_ref[...], 0.0)
        hb = jnp.maximum(
            jnp.dot(hb, w2_ref[...], preferred_element_type=jnp.float32)
            + b2_ref[...], 0.0)

        rows = i * blk + lax.broadcasted_iota(jnp.int32, (blk, 1), 0)
        valid = rows < n_nodes
        hm = jnp.where(valid, hb, -jnp.inf)
        bm = jnp.max(hm, axis=0, keepdims=True)
        m_new = jnp.maximum(m_acc[...], bm)
        contrib = jnp.where(valid, jnp.exp(hb - m_new), 0.0)
        s_acc[...] = (s_acc[...] * jnp.exp(m_acc[...] - m_new)
                      + jnp.sum(contrib, axis=0, keepdims=True))
        m_acc[...] = m_new

        @pl.when(i == nblk - 1)
        def _():
            final = 1.0 / s_acc[...]
            o_ref[...] = jnp.dot(final, wr_ref[...],
                                 preferred_element_type=jnp.float32) + br_ref[...]

    full = lambda shape: pl.BlockSpec(shape, lambda i: (0, 0))
    return pl.pallas_call(
        body,
        grid=(nblk,),
        in_specs=[
            pl.BlockSpec((blk, d), lambda i: (i, 0)),
            pl.BlockSpec((blk, d), lambda i: (i, 0)),
            full((d, h)), full((d, h)), full((1, h)),
            full((h, h)), full((1, h)),
            full((h, h)), full((1, h)),
            full((h, h)), full((1, h)),
        ],
        out_specs=full((1, h)),
        out_shape=jax.ShapeDtypeStruct((1, h), jnp.float32),
        scratch_shapes=[
            pltpu.VMEM((1, h), jnp.float32),
            pltpu.VMEM((1, h), jnp.float32),
        ],
    )(x_pad, agg_pad, W_self, W_neigh, b_conv2, W1, b12, W2, b22,
      Wr_pad, br_pad)


def kernel(x, edge_index, W_pool, b_pool, W_self, W_neigh, b_conv,
           W1, b1, W2, b2, Wr, br):
    n, d = x.shape
    h = W_self.shape[1]
    out_dim = Wr.shape[1]

    npt = _round_up((n + NW - 1) // NW, 8)
    n_pad = NW * npt
    blk = 512
    n_pad = _round_up(n_pad, blk)
    npt_total = n_pad

    x_pad = jnp.pad(x, ((0, n_pad - n), (0, 0)))
    src = edge_index[0]
    dst = edge_index[1]

    h_pool = _tc_pool(x_pad, W_pool, b_pool.reshape(1, d))
    agg = _sc_segment_max(h_pool, src, dst, n, npt)
    if agg.shape[0] != n_pad:
        agg = jnp.pad(agg, ((0, n_pad - agg.shape[0]), (0, 0)))

    Wr_pad = jnp.pad(Wr, ((0, 0), (0, h - out_dim)))
    br_pad = jnp.pad(br, ((0, h - out_dim))).reshape(1, h)
    out = _tc_readout(x_pad, agg, W_self, W_neigh, b_conv.reshape(1, h),
                      W1, b1.reshape(1, h), W2, b2.reshape(1, h),
                      Wr_pad, br_pad, n, blk)
    return out[:, :out_dim]


# speedup vs baseline: 2.2340x; 1.0779x over previous
import functools

import jax
import jax.numpy as jnp
from jax import lax
from jax.experimental import pallas as pl
from jax.experimental.pallas import tpu as pltpu
from jax.experimental.pallas import tpu_sc as plsc

NC = 2
NS = 16
NW = NC * NS
LANES = 16
G = 32


def _round_up(a, b):
    return (a + b - 1) // b * b


def _pick_chunk(E):
    for c in range(8192, 63, -64):
        if E % c == 0:
            return c
    for c in range(8192, 15, -16):
        if E % c == 0:
            return c
    return 16


def _tc_pool(x_pad, W_pool, b_pool2):
    n_pad = x_pad.shape[0]
    d = x_pad.shape[1]

    def body(x_ref, w_ref, b_ref, o_ref):
        acc = jnp.dot(x_ref[...], w_ref[...], preferred_element_type=jnp.float32)
        o_ref[...] = jnp.maximum(acc + b_ref[...], 0.0)

    return pl.pallas_call(
        body,
        out_shape=jax.ShapeDtypeStruct((n_pad, d), jnp.float32),
    )(x_pad, W_pool, b_pool2)


def _sc_segment_max(h_pool, src, dst, n_nodes, npt):
    E = src.shape[0]
    d = h_pool.shape[1]
    dch = d // LANES
    chunk = _pick_chunk(E)
    nchunks = E // chunk
    nvreg = chunk // LANES
    sel_cap = chunk + 2 * G

    mesh = plsc.VectorSubcoreMesh(
        core_axis_name="c", subcore_axis_name="s", num_cores=NC, num_subcores=NS
    )

    @functools.partial(
        pl.kernel,
        out_type=jax.ShapeDtypeStruct((NW, npt, d), jnp.float32),
        mesh=mesh,
        compiler_params=pltpu.CompilerParams(needs_layout_passes=False),
        scratch_types=[
            pltpu.VMEM((npt + 1, d), jnp.float32),
            pltpu.VMEM((chunk,), jnp.int32),
            pltpu.VMEM((chunk,), jnp.int32),
            pltpu.VMEM((chunk,), jnp.int32),
            pltpu.VMEM((chunk,), jnp.int32),
            pltpu.VMEM((sel_cap,), jnp.int32),
            pltpu.VMEM((sel_cap,), jnp.int32),
            pltpu.VMEM((G, d), jnp.float32),
            pltpu.VMEM((G, d), jnp.float32),
            pltpu.SemaphoreType.DMA,
            pltpu.SemaphoreType.DMA,
            pltpu.SemaphoreType.DMA,
            pltpu.SemaphoreType.DMA,
        ],
    )
    def k(hpool_hbm, src_hbm, dst_hbm, agg_hbm,
          agg_l, dst_c0, dst_c1, src_c0, src_c1, sel_s, sel_l, msg0, msg1,
          semc0, semc1, semg0, semg1):
        wid = lax.axis_index("s") * NC + lax.axis_index("c")
        lo = wid * npt
        lane = lax.iota(jnp.int32, LANES)
        zf = jnp.zeros((LANES,), jnp.float32)
        dst_c = [dst_c0, dst_c1]
        src_c = [src_c0, src_c1]
        msg = [msg0, msg1]
        semc = [semc0, semc1]
        semg = [semg0, semg1]

        def zrow(r, _):
            for cc in range(dch):
                agg_l[r, pl.ds(cc * LANES, LANES)] = zf
            return 0
        lax.fori_loop(0, npt + 1, zrow, 0)

        def fire_chunk(c, k_):
            pltpu.async_copy(dst_hbm.at[pl.ds(c * chunk, chunk)],
                             dst_c[k_], semc[k_])
            pltpu.async_copy(src_hbm.at[pl.ds(c * chunk, chunk)],
                             src_c[k_], semc[k_])

        def wait_chunk(k_):
            pltpu.make_async_copy(dst_hbm.at[pl.ds(0, chunk)],
                                  dst_c[k_], semc[k_]).wait()
            pltpu.make_async_copy(src_hbm.at[pl.ds(0, chunk)],
                                  src_c[k_], semc[k_]).wait()

        def fire_grp(g, b):
            pltpu.async_copy(hpool_hbm.at[sel_s.at[pl.ds(g * G, G)]],
                             msg[b], semg[b])

        def wait_grp(b):
            pltpu.make_async_copy(hpool_hbm.at[sel_s.at[pl.ds(0, G)]],
                                  msg[b], semg[b]).wait()

        def accum_grp(g, b):
            for half in range(G // LANES):
                lv = sel_l[pl.ds(g * G + half * LANES, LANES)]
                for j in range(LANES):
                    loc = jnp.max(jnp.where(lane == j, lv, 0))
                    row = half * LANES + j
                    for cc in range(dch):
                        sl = pl.ds(cc * LANES, LANES)
                        agg_l[loc, sl] = jnp.maximum(
                            agg_l[loc, sl], msg[b][row, sl]
                        )

        def process_groups(ngroups, wp_keep):
            @pl.when(ngroups > 0)
            def _():
                fire_grp(0, 0)

            def pair(p, _):
                g0 = 2 * p
                g1 = 2 * p + 1

                @pl.when(g1 < ngroups)
                def _():
                    fire_grp(g1, 1)
                wait_grp(0)
                accum_grp(g0, 0)

                @pl.when(g1 < ngroups)
                def _():
                    @pl.when(g1 + 1 < ngroups)
                    def _():
                        fire_grp(g1 + 1, 0)
                    wait_grp(1)
                    accum_grp(g1, 1)
                return 0
            lax.fori_loop(0, (ngroups + 1) // 2, pair, 0)

            @pl.when(ngroups > 0)
            def _():
                for half in range(G // LANES):
                    svec = sel_s[pl.ds(ngroups * G + half * LANES, LANES)]
                    lvec = sel_l[pl.ds(ngroups * G + half * LANES, LANES)]
                    sel_s[pl.ds(half * LANES, LANES)] = svec
                    sel_l[pl.ds(half * LANES, LANES)] = lvec
            return wp_keep - ngroups * G

        unpt = jnp.uint32(npt)

        def per_chunk(c, wp):
            k_ = lax.rem(c, 2)
            @pl.when(k_ == 0)
            def _():
                wait_chunk(0)

            @pl.when(k_ == 1)
            def _():
                wait_chunk(1)

            @pl.when((c + 1 < nchunks) & (k_ == 0))
            def _():
                fire_chunk(c + 1, 1)

            @pl.when((c + 1 < nchunks) & (k_ == 1))
            def _():
                fire_chunk(c + 1, 0)

            def sel_body(v, wpv, kb):
                dv = dst_c[kb][pl.ds(v * LANES, LANES)]
                loc = dv - lo
                m = loc.astype(jnp.uint32) < unpt
                cv = plsc.cumsum(jnp.where(m, 1, 0))
                pos = wpv + cv
                sv = src_c[kb][pl.ds(v * LANES, LANES)]
                plsc.store_scatter(sel_s, [pos], sv, mask=m)
                plsc.store_scatter(sel_l, [pos], loc, mask=m)
                return wpv + plsc.all_reduce_population_count(m)

            def run_sel(wpv, kb):
                def quad(q, w):
                    for u in range(4):
                        w = sel_body(4 * q + u, w, kb)
                    return w
                return lax.fori_loop(0, nvreg // 4, quad, wpv)

            wpv = lax.cond(k_ == 0,
                           lambda w: run_sel(w, 0),
                           lambda w: run_sel(w, 1), wp)
            wp_s = jnp.max(wpv) + 1
            rem = process_groups(wp_s // G, wp_s)
            return jnp.full((LANES,), rem - 1, jnp.int32)

        fire_chunk(0, 0)
        wpv_fin = lax.fori_loop(0, nchunks, per_chunk,
                                jnp.full((LANES,), -1, jnp.int32))
        wp = jnp.max(wpv_fin) + 1

        @pl.when(wp > 0)
        def _():
            for half in range(G // LANES):
                pidx = wp + half * LANES + lane
                mpad = pidx < G
                plsc.store_scatter(sel_s, [pidx], jnp.zeros((LANES,), jnp.int32),
                                   mask=mpad)
                plsc.store_scatter(sel_l, [pidx],
                                   jnp.full((LANES,), npt, jnp.int32), mask=mpad)
            process_groups(1, wp)

        pltpu.sync_copy(agg_l.at[pl.ds(0, npt)], agg_hbm.at[wid])

    out = k(h_pool, src, dst)
    return out.reshape(NW * npt, d)


def _tc_readout(x_pad, agg_pad, W_self, W_neigh, b_conv2, W1, b12, W2, b22,
                Wr_pad, br_pad, n_nodes, blk):
    n_pad = x_pad.shape[0]
    d = x_pad.shape[1]
    h = W_self.shape[1]
    nblk = n_pad // blk

    def body(x_ref, a_ref, ws_ref, wn_ref, bc_ref, w1_ref, b1_ref,
             w2_ref, b2_ref, wr_ref, br_ref, o_ref, m_acc, s_acc):
        i = pl.program_id(0)

        @pl.when(i == 0)
        def _():
            m_acc[...] = jnp.full((1, h), -jnp.inf, jnp.float32)
            s_acc[...] = jnp.zeros((1, h), jnp.float32)

        hb = jnp.dot(x_ref[...], ws_ref[...], preferred_element_type=jnp.float32)
        hb += jnp.dot(a_ref[...], wn_ref[...], preferred_element_type=jnp.float32)
        hb = jnp.maximum(hb + bc_ref[...], 0.0)
        hb = jnp.maximum(
            jnp.dot(hb, w1_ref[...], preferred_element_type=jnp.float32)
            + b1_ref[...], 0.0)
        hb = jnp.maximum(
            jnp.dot(hb, w2_ref[...], preferred_element_type=jnp.float32)
            + b2_ref[...], 0.0)

        rows = i * blk + lax.broadcasted_iota(jnp.int32, (blk, 1), 0)
        valid = rows < n_nodes
        hm = jnp.where(valid, hb, -jnp.inf)
        bm = jnp.max(hm, axis=0, keepdims=True)
        m_new = jnp.maximum(m_acc[...], bm)
        contrib = jnp.where(valid, jnp.exp(hb - m_new), 0.0)
        s_acc[...] = (s_acc[...] * jnp.exp(m_acc[...] - m_new)
                      + jnp.sum(contrib, axis=0, keepdims=True))
        m_acc[...] = m_new

        @pl.when(i == nblk - 1)
        def _():
            final = 1.0 / s_acc[...]
            o_ref[...] = jnp.dot(final, wr_ref[...],
                                 preferred_element_type=jnp.float32) + br_ref[...]

    full = lambda shape: pl.BlockSpec(shape, lambda i: (0, 0))
    return pl.pallas_call(
        body,
        grid=(nblk,),
        in_specs=[
            pl.BlockSpec((blk, d), lambda i: (i, 0)),
            pl.BlockSpec((blk, d), lambda i: (i, 0)),
            full((d, h)), full((d, h)), full((1, h)),
            full((h, h)), full((1, h)),
            full((h, h)), full((1, h)),
            full((h, h)), full((1, h)),
        ],
        out_specs=full((1, h)),
        out_shape=jax.ShapeDtypeStruct((1, h), jnp.float32),
        scratch_shapes=[
            pltpu.VMEM((1, h), jnp.float32),
            pltpu.VMEM((1, h), jnp.float32),
        ],
    )(x_pad, agg_pad, W_self, W_neigh, b_conv2, W1, b12, W2, b22,
      Wr_pad, br_pad)


def kernel(x, edge_index, W_pool, b_pool, W_self, W_neigh, b_conv,
           W1, b1, W2, b2, Wr, br):
    n, d = x.shape
    h = W_self.shape[1]
    out_dim = Wr.shape[1]

    npt = _round_up((n + NW - 1) // NW, 8)
    n_pad = NW * npt
    blk = 512
    n_pad = _round_up(n_pad, blk)
    npt_total = n_pad

    x_pad = jnp.pad(x, ((0, n_pad - n), (0, 0)))
    src = edge_index[0]
    dst = edge_index[1]

    h_pool = _tc_pool(x_pad, W_pool, b_pool.reshape(1, d))
    agg = _sc_segment_max(h_pool, src, dst, n, npt)
    if agg.shape[0] != n_pad:
        agg = jnp.pad(agg, ((0, n_pad - agg.shape[0]), (0, 0)))

    Wr_pad = jnp.pad(Wr, ((0, 0), (0, h - out_dim)))
    br_pad = jnp.pad(br, ((0, h - out_dim))).reshape(1, h)
    out = _tc_readout(x_pad, agg, W_self, W_neigh, b_conv.reshape(1, h),
                      W1, b1.reshape(1, h), W2, b2.reshape(1, h),
                      Wr_pad, br_pad, n, blk)
    return out[:, :out_dim]
